# Initial kernel scaffold; baseline (speedup 1.0000x reference)
import jax
import jax.numpy as jnp
from jax.experimental import pallas as pl


def kernel(v_n, f_decode, agenttypes, ctrs, edge_index, at_W, at_b, ep_l1_W, ep_l1_b, ep_n1_g, ep_n1_b, ep_l2_W, ep_l2_b, ep_n2_g, ep_n2_b, ep_t_W, ep_t_b, ep_tn_g, ep_tn_b, fo_l1_W, fo_l1_b, fo_n1_g, fo_n1_b, fo_l2_W, fo_l2_b, fo_n2_g, fo_n2_b, fo_t_W, fo_t_b, fo_tn_g, fo_tn_b, fc1_W, fc2_W, fc3_W, gru_W_ih, gru_W_hh, gru_b_ih, gru_b_hh, fo_out_W, fo_out_b):
    raise NotImplementedError("write your pallas kernel here")


# speedup vs baseline: 6.2412x; 6.2412x over previous
import functools

import jax
import jax.numpy as jnp
from jax import lax
from jax.experimental import pallas as pl
from jax.experimental.pallas import tpu as pltpu
from jax.experimental.pallas import tpu_sc as plsc

N, E, H, M, T, NA = 5000, 50000, 128, 6, 30, 2
G = 32
E_PAD = 51200
RT = E_PAD * M
NACC = 30080
BE = 512
BR = 1200
KB = 64
SB = 128
NW = 32
GATHER_BATCHES = E_PAD // (NW * KB)
ROWS_PER_W = E_PAD // NW
SC_ROWS_PER_TILE = RT // 16
SC_BATCHES = SC_ROWS_PER_TILE // SB
ACC_PER_TILE = NACC // 16

_mesh = plsc.VectorSubcoreMesh(core_axis_name="c", subcore_axis_name="s")


@functools.partial(
    pl.kernel, mesh=_mesh,
    out_type=[jax.ShapeDtypeStruct((E_PAD, 384), jnp.float32),
              jax.ShapeDtypeStruct((E_PAD, 128), jnp.float32)],
    scratch_types=[pltpu.VMEM((KB,), jnp.int32),
                   pltpu.VMEM((KB,), jnp.int32),
                   pltpu.VMEM((KB, 384), jnp.float32),
                   pltpu.VMEM((KB, 128), jnp.float32),
                   pltpu.SemaphoreType.DMA,
                   pltpu.SemaphoreType.DMA],
)
def _sc_gather(tsrc, tdst, sidx, didx, out_src, out_dst,
               idx_v, idx2_v, rows_v, rows2_v, sem1, sem2):
    wid = lax.axis_index("s") * 2 + lax.axis_index("c")

    def body(b, carry):
        base = wid * ROWS_PER_W + b * KB
        pltpu.sync_copy(sidx.at[pl.ds(base, KB)], idx_v)
        pltpu.sync_copy(didx.at[pl.ds(base, KB)], idx2_v)
        pltpu.async_copy(tsrc.at[idx_v], rows_v, sem1).wait()
        pltpu.sync_copy(rows_v, out_src.at[pl.ds(base, KB)])
        pltpu.async_copy(tdst.at[idx2_v], rows2_v, sem2).wait()
        pltpu.sync_copy(rows2_v, out_dst.at[pl.ds(base, KB)])
        return carry

    lax.fori_loop(0, GATHER_BATCHES, body, 0)


@functools.partial(
    pl.kernel, mesh=_mesh,
    out_type=jax.ShapeDtypeStruct((4, NACC, 32), jnp.float32),
    scratch_types=[pltpu.VMEM((SB,), jnp.int32),
                   pltpu.VMEM((SB, 32), jnp.float32),
                   pltpu.VMEM_SHARED((NACC, 32), jnp.float32)],
)
def _sc_scatter(wz, dstm, zer, out, idx_v, rows_v, acc_sh):
    sc = lax.axis_index("c")
    sid = lax.axis_index("s")
    zbase = sid * ACC_PER_TILE
    for ql in range(2):
        q = sc * 2 + ql
        pltpu.sync_copy(zer.at[pl.ds(zbase, ACC_PER_TILE)],
                        acc_sh.at[pl.ds(zbase, ACC_PER_TILE)])
        plsc.subcore_barrier()

        def body(b, carry):
            base = sid * SC_ROWS_PER_TILE + b * SB
            pltpu.sync_copy(dstm.at[pl.ds(base, SB)], idx_v)
            pltpu.sync_copy(wz.at[q, pl.ds(base, SB)], rows_v)
            pltpu.sync_copy(rows_v, acc_sh.at[idx_v], add=True)
            return carry

        lax.fori_loop(0, SC_BATCHES, body, 0)
        plsc.subcore_barrier()
        pltpu.sync_copy(acc_sh.at[pl.ds(zbase, ACC_PER_TILE)],
                        out.at[q, pl.ds(zbase, ACC_PER_TILE)])
        plsc.subcore_barrier()


def _gn(x, A, g, b, eps=1e-5):
    mu = jnp.dot(x, A, preferred_element_type=jnp.float32)
    var = jnp.dot(x * x, A, preferred_element_type=jnp.float32) - mu * mu
    return ((x - mu) * jax.lax.rsqrt(var + eps)) * g + b


def _elu(x):
    return jnp.where(x > 0, x, jnp.exp(jnp.minimum(x, 0.0)) - 1.0)


def _mm(a, b):
    return jnp.dot(a, b, preferred_element_type=jnp.float32)


def _sn_body(v_ref, q_ref, o_ref):
    o_ref[...] = _mm(v_ref[...], q_ref[...])


def _edge_body(gsrc_ref, gdst_ref, w1_ref, b1_ref, wc1_ref, n1g_ref, n1b_ref,
               w2_ref, b2_ref, n2g_ref, n2b_ref, wt_ref, bt_ref, wct_ref,
               tng_ref, tnb_ref, atw_ref, atb_ref, fc1_ref, w3_ref, A_ref,
               zm_ref, et_ref):
    A = A_ref[...]
    fd = jnp.concatenate(
        [gsrc_ref[:, m * 60:(m + 1) * 60] for m in range(M)], axis=0)
    ctrs_d = gdst_ref[:, 2:4]
    ctr1 = _mm(ctrs_d, wc1_ref[...])
    ctrt = _mm(ctrs_d, wct_ref[...])
    ctr1_b = jnp.concatenate([ctr1] * M, axis=0)
    ctrt_b = jnp.concatenate([ctrt] * M, axis=0)

    u1 = _mm(fd, w1_ref[...]) - ctr1_b + b1_ref[...]
    u1 = _gn(u1, A, n1g_ref[...], n1b_ref[...])
    u1 = _elu(u1)
    u1 = _mm(u1, w2_ref[...]) + b2_ref[...]
    u1 = _gn(u1, A, n2g_ref[...], n2b_ref[...])
    ut = _mm(fd, wt_ref[...]) - ctrt_b + bt_ref[...]
    ut = _gn(ut, A, tng_ref[...], tnb_ref[...])
    pf = _elu(u1 + ut)

    types = jnp.concatenate([gsrc_ref[:, 360:362], gdst_ref[:, 0:2]], axis=1)
    at = _mm(types, atw_ref[...]) + atb_ref[...]
    at_b = jnp.concatenate([at] * M, axis=0)
    z_m = _mm(pf + at_b, fc1_ref[...])
    zm_ref[...] = z_m

    s_nd = jnp.concatenate(
        [gdst_ref[:, 4 + m:5 + m] for m in range(M)], axis=0)
    e_raw = _mm(z_m, w3_ref[...]) + s_nd
    et_ref[...] = jnp.where(e_raw > 0, e_raw, 0.2 * e_raw)


SW = 512
NBLK = E_PAD // SW
_NEG = -3.4e38


def _seg_scan_block(v, fl, pre, op, fill, reverse):
    k = 1
    while k < SW:
        if not reverse:
            vs = jnp.concatenate(
                [jnp.full((M, k), fill, jnp.float32), v[:, :-k]], axis=1)
            fs = jnp.concatenate(
                [jnp.full((M, k), 1.0, jnp.float32), fl[:, :-k]], axis=1)
            ps = jnp.concatenate(
                [jnp.full((M, k), 0.0, jnp.float32), pre[:, :-k]], axis=1)
        else:
            vs = jnp.concatenate(
                [v[:, k:], jnp.full((M, k), fill, jnp.float32)], axis=1)
            fs = jnp.concatenate(
                [fl[:, k:], jnp.full((M, k), 1.0, jnp.float32)], axis=1)
            ps = jnp.concatenate(
                [pre[:, k:], jnp.full((M, k), 0.0, jnp.float32)], axis=1)
        v = jnp.where(fl > 0.5, v, op(v, vs))
        fl = jnp.maximum(fl, fs)
        pre = jnp.maximum(pre, ps)
        k *= 2
    return v, pre


def _scan_fwd_max(e_ref, f_ref, o_ref, carry_ref):
    @pl.when(pl.program_id(0) == 0)
    def _():
        carry_ref[...] = jnp.full((M, 128), _NEG, jnp.float32)
    f = jnp.broadcast_to(f_ref[...], (M, SW))
    v, pre = _seg_scan_block(e_ref[...], f, f, jnp.maximum, _NEG, False)
    v = jnp.where(pre > 0.5, v, jnp.maximum(v, carry_ref[:, 0:1]))
    carry_ref[:, 0:1] = v[:, SW - 1:SW]
    o_ref[...] = v


def _scan_bwd_max_ex(e_ref, f_ref, mf_ref, o_ref, carry_ref):
    @pl.when(pl.program_id(0) == 0)
    def _():
        carry_ref[...] = jnp.full((M, 128), _NEG, jnp.float32)
    f = jnp.broadcast_to(f_ref[...], (M, SW))
    v0 = e_ref[...]
    v, suf = _seg_scan_block(v0, f, f, jnp.maximum, _NEG, True)
    v = jnp.where(suf > 0.5, v, jnp.maximum(v, carry_ref[:, 0:1]))
    carry_ref[:, 0:1] = v[:, 0:1]
    o_ref[...] = jnp.exp(v0 - jnp.maximum(v, mf_ref[...]))


def _scan_fwd_sum(x_ref, f_ref, o_ref, carry_ref):
    @pl.when(pl.program_id(0) == 0)
    def _():
        carry_ref[...] = jnp.zeros((M, 128), jnp.float32)
    f = jnp.broadcast_to(f_ref[...], (M, SW))
    v, pre = _seg_scan_block(x_ref[...], f, f, jnp.add, 0.0, False)
    v = jnp.where(pre > 0.5, v, v + carry_ref[:, 0:1])
    carry_ref[:, 0:1] = v[:, SW - 1:SW]
    o_ref[...] = v


def _scan_bwd_sum_alpha(x_ref, f_ref, sf_ref, o_ref, carry_ref):
    @pl.when(pl.program_id(0) == 0)
    def _():
        carry_ref[...] = jnp.zeros((M, 128), jnp.float32)
    f = jnp.broadcast_to(f_ref[...], (M, SW))
    x = x_ref[...]
    v, suf = _seg_scan_block(x, f, f, jnp.add, 0.0, True)
    v = jnp.where(suf > 0.5, v, v + carry_ref[:, 0:1])
    carry_ref[:, 0:1] = v[:, 0:1]
    o_ref[...] = x / (v + sf_ref[...] - x)


def _weight_body(zm_ref, a_ref, o_ref):
    wz = zm_ref[...] * a_ref[...]
    o_ref[0] = wz[:, 0:32]
    o_ref[1] = wz[:, 32:64]
    o_ref[2] = wz[:, 64:96]
    o_ref[3] = wz[:, 96:128]


def _node_body(agg_ref, h_ref, wih_ref, bih_ref, whh_ref, bhh_ref,
               l1a_ref, l1bt_ref, l1b_ref, n1g_ref, n1b_ref,
               w2_ref, b2_ref, n2g_ref, n2b_ref,
               ta_ref, tbt_ref, tb_ref, tng_ref, tnb_ref,
               wo_ref, bo_ref, ctr_ref, A_ref,
               hn_ref, d_ref):
    A = A_ref[...]
    gi = _mm(agg_ref[...], wih_ref[...]) + bih_ref[...]
    h = h_ref[...]
    gh = _mm(h, whh_ref[...]) + bhh_ref[...]
    r = jax.nn.sigmoid(gi[:, :H] + gh[:, :H])
    z = jax.nn.sigmoid(gi[:, H:2 * H] + gh[:, H:2 * H])
    n = jnp.tanh(gi[:, 2 * H:] + r * gh[:, 2 * H:])
    h_new = (1.0 - z) * n + z * h
    hn_ref[...] = h_new

    o1 = _mm(h_new, l1a_ref[...]) + l1bt_ref[...] + l1b_ref[...]
    o1 = _gn(o1, A, n1g_ref[...], n1b_ref[...])
    o1 = _elu(o1)
    o1 = _mm(o1, w2_ref[...]) + b2_ref[...]
    o1 = _gn(o1, A, n2g_ref[...], n2b_ref[...])
    ot = _mm(h_new, ta_ref[...]) + tbt_ref[...] + tb_ref[...]
    ot = _gn(ot, A, tng_ref[...], tnb_ref[...])
    dec = _elu(o1 + ot)
    dests = _mm(dec, wo_ref[...]) + bo_ref[...]
    d_ref[...] = dests + ctr_ref[...]


def _rep(shape):
    nd = len(shape)
    return pl.BlockSpec(shape, lambda i: (0,) * nd)


def kernel(v_n, f_decode, agenttypes, ctrs, edge_index, at_W, at_b, ep_l1_W,
           ep_l1_b, ep_n1_g, ep_n1_b, ep_l2_W, ep_l2_b, ep_n2_g, ep_n2_b,
           ep_t_W, ep_t_b, ep_tn_g, ep_tn_b, fo_l1_W, fo_l1_b, fo_n1_g,
           fo_n1_b, fo_l2_W, fo_l2_b, fo_n2_g, fo_n2_b, fo_t_W, fo_t_b,
           fo_tn_g, fo_tn_b, fc1_W, fc2_W, fc3_W, gru_W_ih, gru_W_hh,
           gru_b_ih, gru_b_hh, fo_out_W, fo_out_b):
    f32 = jnp.float32
    src, dst = edge_index[0], edge_index[1]
    order = jnp.argsort(dst)
    src_s = jnp.concatenate([src[order], jnp.zeros((E_PAD - E,), jnp.int32)])
    dst_s = jnp.concatenate([dst[order], jnp.full((E_PAD - E,), N, jnp.int32)])

    Wc1 = ep_l1_W.reshape(H, T, 2).sum(1)
    Wct = ep_t_W.reshape(H, T, 2).sum(1)
    q = fc2_W.T @ fc3_W[0, H:]
    w3m = fc3_W[0, :H].reshape(H, 1)
    A = jnp.kron(jnp.eye(G, dtype=f32), jnp.full((H // G, H // G), G / H, f32))

    vflat = v_n.reshape(N * M, H)
    s_n = pl.pallas_call(
        _sn_body, grid=(25,),
        in_specs=[pl.BlockSpec((BR, H), lambda i: (i, 0)), _rep((H, 1))],
        out_specs=pl.BlockSpec((BR, 1), lambda i: (i, 0)),
        out_shape=jax.ShapeDtypeStruct((N * M, 1), f32),
    )(vflat, q.reshape(H, 1))

    tbl_src = jnp.concatenate(
        [f_decode.reshape(N, 360), agenttypes, jnp.zeros((N, 22), f32)], axis=1)
    tbl_dst = jnp.concatenate(
        [agenttypes, ctrs, s_n.reshape(N, M), jnp.zeros((N, 118), f32)], axis=1)
    tbl_dst = jnp.concatenate([tbl_dst, jnp.zeros((8, 128), f32)], axis=0)
    g_src, g_dst = _sc_gather(tbl_src, tbl_dst, src_s, dst_s)

    bspec = [
        pl.BlockSpec((BE, 384), lambda i: (i, 0)),
        pl.BlockSpec((BE, 128), lambda i: (i, 0)),
        _rep((60, H)), _rep((1, H)), _rep((2, H)), _rep((1, H)), _rep((1, H)),
        _rep((H, H)), _rep((1, H)), _rep((1, H)), _rep((1, H)),
        _rep((60, H)), _rep((1, H)), _rep((2, H)), _rep((1, H)), _rep((1, H)),
        _rep((4, H)), _rep((1, H)), _rep((H, H)), _rep((H, 1)), _rep((H, H)),
    ]
    z_m, e_rows = pl.pallas_call(
        _edge_body, grid=(E_PAD // BE,),
        in_specs=bspec,
        out_specs=[pl.BlockSpec((BE * M, H), lambda i: (i, 0)),
                   pl.BlockSpec((BE * M, 1), lambda i: (i, 0))],
        out_shape=[jax.ShapeDtypeStruct((RT, H), f32),
                   jax.ShapeDtypeStruct((RT, 1), f32)],
    )(g_src, g_dst, ep_l1_W.T, ep_l1_b.reshape(1, H), Wc1.T,
      ep_n1_g.reshape(1, H), ep_n1_b.reshape(1, H), ep_l2_W.T,
      ep_l2_b.reshape(1, H), ep_n2_g.reshape(1, H), ep_n2_b.reshape(1, H),
      ep_t_W.T, ep_t_b.reshape(1, H), Wct.T, ep_tn_g.reshape(1, H),
      ep_tn_b.reshape(1, H), at_W.T, at_b.reshape(1, H), fc1_W.T, w3m, A)

    NB = E_PAD // BE
    e_t = e_rows.reshape(NB, M, BE).transpose(1, 0, 2).reshape(M, E_PAD)
    sflag = jnp.concatenate(
        [jnp.ones((1,), jnp.float32),
         (dst_s[1:] != dst_s[:-1]).astype(jnp.float32)]).reshape(1, E_PAD)
    eflag = jnp.concatenate(
        [(dst_s[:-1] != dst_s[1:]).astype(jnp.float32),
         jnp.ones((1,), jnp.float32)]).reshape(1, E_PAD)

    blk_f = pl.BlockSpec((M, SW), lambda i: (0, i))
    fblk_f = pl.BlockSpec((1, SW), lambda i: (0, i))
    blk_r = pl.BlockSpec((M, SW), lambda i: (0, NBLK - 1 - i))
    fblk_r = pl.BlockSpec((1, SW), lambda i: (0, NBLK - 1 - i))
    scr = [pltpu.VMEM((M, 128), f32)]
    shp = jax.ShapeDtypeStruct((M, E_PAD), f32)

    mf = pl.pallas_call(_scan_fwd_max, grid=(NBLK,),
                        in_specs=[blk_f, fblk_f], out_specs=blk_f,
                        out_shape=shp, scratch_shapes=scr)(e_t, sflag)
    ex = pl.pallas_call(_scan_bwd_max_ex, grid=(NBLK,),
                        in_specs=[blk_r, fblk_r, blk_r], out_specs=blk_r,
                        out_shape=shp, scratch_shapes=scr)(e_t, eflag, mf)
    sfw = pl.pallas_call(_scan_fwd_sum, grid=(NBLK,),
                         in_specs=[blk_f, fblk_f], out_specs=blk_f,
                         out_shape=shp, scratch_shapes=scr)(ex, sflag)
    alpha = pl.pallas_call(_scan_bwd_sum_alpha, grid=(NBLK,),
                           in_specs=[blk_r, fblk_r, blk_r], out_specs=blk_r,
                           out_shape=shp, scratch_shapes=scr)(ex, eflag, sfw)
    a_rows = alpha.reshape(M, NB, BE).transpose(1, 0, 2).reshape(RT, 1)

    wz = pl.pallas_call(
        _weight_body, grid=(E_PAD // BE,),
        in_specs=[pl.BlockSpec((BE * M, H), lambda i: (i, 0)),
                  pl.BlockSpec((BE * M, 1), lambda i: (i, 0))],
        out_specs=pl.BlockSpec((4, BE * M, 32), lambda i: (0, i, 0)),
        out_shape=jax.ShapeDtypeStruct((4, RT, 32), f32),
    )(z_m, a_rows)

    mcol = jnp.arange(M, dtype=jnp.int32)[None, :, None]
    dst_blk = dst_s.reshape(NB, 1, BE)
    dstm = jnp.where(dst_blk >= N, N * M,
                     dst_blk * M + mcol).reshape(RT)
    wzf = jnp.concatenate([wz[0], wz[1], wz[2], wz[3]], axis=1)
    agg = jnp.zeros((NACC, H), f32).at[dstm].add(wzf)[:N * M]

    l1bt = jnp.tile(fo_l1_W[:, H:].T, (BR // M, 1))
    tbt = jnp.tile(fo_t_W[:, H:].T, (BR // M, 1))
    ctr_rows = jnp.tile(jnp.repeat(ctrs, M, axis=0), (1, T))
    nspec = [
        pl.BlockSpec((BR, H), lambda i: (i, 0)),
        pl.BlockSpec((BR, H), lambda i: (i, 0)),
        _rep((H, 3 * H)), _rep((1, 3 * H)), _rep((H, 3 * H)), _rep((1, 3 * H)),
        _rep((H, H)), _rep((BR, H)), _rep((1, H)), _rep((1, H)), _rep((1, H)),
        _rep((H, H)), _rep((1, H)), _rep((1, H)), _rep((1, H)),
        _rep((H, H)), _rep((BR, H)), _rep((1, H)), _rep((1, H)), _rep((1, H)),
        _rep((H, 2 * T)), _rep((1, 2 * T)),
        pl.BlockSpec((BR, 2 * T), lambda i: (i, 0)), _rep((H, H)),
    ]
    h_new, dests = pl.pallas_call(
        _node_body, grid=(N * M // BR,),
        in_specs=nspec,
        out_specs=[pl.BlockSpec((BR, H), lambda i: (i, 0)),
                   pl.BlockSpec((BR, 2 * T), lambda i: (i, 0))],
        out_shape=[jax.ShapeDtypeStruct((N * M, H), f32),
                   jax.ShapeDtypeStruct((N * M, 2 * T), f32)],
    )(agg, vflat, gru_W_ih.T, gru_b_ih.reshape(1, 3 * H), gru_W_hh.T,
      gru_b_hh.reshape(1, 3 * H), fo_l1_W[:, :H].T, l1bt,
      fo_l1_b.reshape(1, H), fo_n1_g.reshape(1, H), fo_n1_b.reshape(1, H),
      fo_l2_W.T, fo_l2_b.reshape(1, H), fo_n2_g.reshape(1, H),
      fo_n2_b.reshape(1, H), fo_t_W[:, :H].T, tbt, fo_t_b.reshape(1, H),
      fo_tn_g.reshape(1, H), fo_tn_b.reshape(1, H), fo_out_W.T,
      fo_out_b.reshape(1, 2 * T), ctr_rows, A)

    return h_new.reshape(N, M, H), dests.reshape(N, M, T, 2)
